# baseline (speedup 1.0000x reference)
import functools

import jax
import jax.numpy as jnp
import numpy as np
from jax import lax
from jax.experimental import pallas as pl
from jax.experimental.pallas import tpu as pltpu
from jax.experimental.pallas import tpu_sc as plsc

NUM_CLASSES = 100000
DIM = 128
BATCH = 4096
NUM_SAMPLED = 1000
S_PAD = 1024

_NW = 32
_TRUE_PER_W = BATCH // _NW
_SAMP_PER_W = S_PAD // _NW


def _log_uniform_prob(ids_f):
    return (jnp.log(ids_f + 2.0) - jnp.log(ids_f + 1.0)) / jnp.log(
        float(NUM_CLASSES) + 1.0
    )


def _draw_sampled_ids():
    ks = jax.random.key(42)
    u = jax.random.uniform(ks, (NUM_SAMPLED,), dtype=jnp.float32)
    ids = jnp.floor(jnp.exp(u * jnp.log(float(NUM_CLASSES) + 1.0))) - 1.0
    return jnp.clip(ids, 0, NUM_CLASSES - 1).astype(jnp.int32)


def _sc_gather(table, idx, n_rows, per_w):
    mesh = plsc.VectorSubcoreMesh(core_axis_name="c", subcore_axis_name="s")

    @functools.partial(
        pl.kernel,
        out_type=jax.ShapeDtypeStruct((n_rows, DIM), jnp.float32),
        mesh=mesh,
        scratch_types=(
            pltpu.VMEM((per_w,), jnp.int32),
            pltpu.VMEM((per_w, DIM), jnp.float32),
            pltpu.SemaphoreType.DMA,
        ),
    )
    def gather_kernel(table_hbm, idx_hbm, out_hbm, idx_v, rows_v, sem):
        wid = lax.axis_index("s") * 2 + lax.axis_index("c")
        base = wid * per_w
        pltpu.sync_copy(idx_hbm.at[pl.ds(base, per_w)], idx_v)
        pltpu.async_copy(table_hbm.at[idx_v], rows_v, sem).wait()
        pltpu.sync_copy(rows_v, out_hbm.at[pl.ds(base, per_w)])

    return gather_kernel(table, idx)


_BLK = 512


def _tc_matmul_body(embed_ref, sampw_ref, lbl_ref, sid_ref, soff_ref, m_ref, z_ref):
    e = embed_ref[...]
    sw = sampw_ref[...]
    s = lax.dot_general(
        e, sw, (((1,), (1,)), ((), ())), preferred_element_type=jnp.float32
    )
    s = s + soff_ref[...]
    hit = lbl_ref[...] == sid_ref[...]
    s = jnp.where(hit, s - 1e9, s)
    m_s = jnp.max(s, axis=1, keepdims=True)
    m_ref[...] = m_s
    z_ref[...] = jnp.sum(jnp.exp(s - m_s), axis=1, keepdims=True)


def _tc_combine_body(embed_ref, truew_ref, lbl_ref, ms_ref, z_ref, out_ref):
    e = embed_ref[...]
    tw = truew_ref[...]
    lf = lbl_ref[...].astype(jnp.float32)
    true_expected = _log_uniform_prob(lf) * float(NUM_SAMPLED)
    t = jnp.sum(e * tw, axis=1, keepdims=True) - jnp.log(true_expected)
    m_s = ms_ref[...]
    z = z_ref[...]
    m = jnp.maximum(m_s, t)
    lse = jnp.log(jnp.exp(t - m) + z * jnp.exp(m_s - m)) + m
    out_ref[...] = lse - t


def kernel(softmax_weights, embed, label_idx, zero_bias):
    del zero_bias
    labels = label_idx.reshape(-1)
    lbl2d = label_idx.reshape(BATCH, 1)

    sampled_ids = _draw_sampled_ids()
    samp_idx_pad = jnp.concatenate(
        [sampled_ids, jnp.zeros((S_PAD - NUM_SAMPLED,), jnp.int32)]
    )
    sid_mask = jnp.concatenate(
        [sampled_ids, jnp.full((S_PAD - NUM_SAMPLED,), -1, jnp.int32)]
    ).reshape(1, S_PAD)
    sampled_expected = _log_uniform_prob(
        sampled_ids.astype(jnp.float32)
    ) * float(NUM_SAMPLED)
    soff = jnp.concatenate(
        [-jnp.log(sampled_expected),
         jnp.full((S_PAD - NUM_SAMPLED,), -1e30, jnp.float32)]
    ).reshape(1, S_PAD)

    samp_w = _sc_gather(softmax_weights, samp_idx_pad, S_PAD, _SAMP_PER_W)
    true_w = _sc_gather(softmax_weights, labels, BATCH, _TRUE_PER_W)

    grid = (BATCH // _BLK,)
    m_s, z_s = pl.pallas_call(
        _tc_matmul_body,
        grid=grid,
        in_specs=[
            pl.BlockSpec((_BLK, DIM), lambda i: (i, 0)),
            pl.BlockSpec((S_PAD, DIM), lambda i: (0, 0)),
            pl.BlockSpec((_BLK, 1), lambda i: (i, 0)),
            pl.BlockSpec((1, S_PAD), lambda i: (0, 0)),
            pl.BlockSpec((1, S_PAD), lambda i: (0, 0)),
        ],
        out_specs=[
            pl.BlockSpec((_BLK, 1), lambda i: (i, 0)),
            pl.BlockSpec((_BLK, 1), lambda i: (i, 0)),
        ],
        out_shape=[
            jax.ShapeDtypeStruct((BATCH, 1), jnp.float32),
            jax.ShapeDtypeStruct((BATCH, 1), jnp.float32),
        ],
    )(embed, samp_w, lbl2d, sid_mask, soff)

    loss = pl.pallas_call(
        _tc_combine_body,
        grid=grid,
        in_specs=[
            pl.BlockSpec((_BLK, DIM), lambda i: (i, 0)),
            pl.BlockSpec((_BLK, DIM), lambda i: (i, 0)),
            pl.BlockSpec((_BLK, 1), lambda i: (i, 0)),
            pl.BlockSpec((_BLK, 1), lambda i: (i, 0)),
            pl.BlockSpec((_BLK, 1), lambda i: (i, 0)),
        ],
        out_specs=pl.BlockSpec((_BLK, 1), lambda i: (i, 0)),
        out_shape=jax.ShapeDtypeStruct((BATCH, 1), jnp.float32),
    )(embed, true_w, lbl2d, m_s, z_s)

    return loss.reshape(-1)


# speedup vs baseline: 1.0756x; 1.0756x over previous
import functools

import jax
import jax.numpy as jnp
import numpy as np
from jax import lax
from jax.experimental import pallas as pl
from jax.experimental.pallas import tpu as pltpu
from jax.experimental.pallas import tpu_sc as plsc

NUM_CLASSES = 100000
DIM = 128
BATCH = 4096
NUM_SAMPLED = 1000
S_PAD = 1024

_NW = 32
_TRUE_PER_W = BATCH // _NW
_SAMP_PER_W = S_PAD // _NW


def _log_uniform_prob(ids_f):
    return (jnp.log(ids_f + 2.0) - jnp.log(ids_f + 1.0)) / jnp.log(
        float(NUM_CLASSES) + 1.0
    )


def _draw_sampled_ids():
    ks = jax.random.key(42)
    u = jax.random.uniform(ks, (NUM_SAMPLED,), dtype=jnp.float32)
    ids = jnp.floor(jnp.exp(u * jnp.log(float(NUM_CLASSES) + 1.0))) - 1.0
    return jnp.clip(ids, 0, NUM_CLASSES - 1).astype(jnp.int32)


def _sc_gather(table, idx, n_rows, per_w):
    mesh = plsc.VectorSubcoreMesh(core_axis_name="c", subcore_axis_name="s")

    @functools.partial(
        pl.kernel,
        out_type=jax.ShapeDtypeStruct((n_rows, DIM), jnp.float32),
        mesh=mesh,
        scratch_types=(
            pltpu.VMEM((per_w,), jnp.int32),
            pltpu.VMEM((per_w, DIM), jnp.float32),
            pltpu.SemaphoreType.DMA,
        ),
    )
    def gather_kernel(table_hbm, idx_hbm, out_hbm, idx_v, rows_v, sem):
        wid = lax.axis_index("s") * 2 + lax.axis_index("c")
        base = wid * per_w
        pltpu.sync_copy(idx_hbm.at[pl.ds(base, per_w)], idx_v)
        pltpu.async_copy(table_hbm.at[idx_v], rows_v, sem).wait()
        pltpu.sync_copy(rows_v, out_hbm.at[pl.ds(base, per_w)])

    return gather_kernel(table, idx)


_BLK = 512


def _read_col(ref, i):
    full = ref[...]
    lane = lax.broadcasted_iota(jnp.int32, full.shape, 1)
    return jnp.sum(jnp.where(lane == i, full, 0), axis=1, keepdims=True)


def _write_col(ref, i, v):
    full = ref[...]
    lane = lax.broadcasted_iota(jnp.int32, full.shape, 1)
    ref[...] = jnp.where(lane == i, v, full)


def _tc_matmul_body(embed_ref, sampw_ref, lbl_ref, sid_ref, soff_ref, m_ref, z_ref):
    i = pl.program_id(0)
    e = embed_ref[...]
    sw = sampw_ref[...]
    s = lax.dot_general(
        e, sw, (((1,), (1,)), ((), ())), preferred_element_type=jnp.float32
    )
    s = s + soff_ref[...]
    hit = _read_col(lbl_ref, i) == sid_ref[...]
    s = jnp.where(hit, s - 1e9, s)
    m_s = jnp.max(s, axis=1, keepdims=True)
    _write_col(m_ref, i, m_s)
    _write_col(z_ref, i, jnp.sum(jnp.exp(s - m_s), axis=1, keepdims=True))


def _tc_combine_body(embed_ref, truew_ref, lbl_ref, ms_ref, z_ref, out_ref):
    i = pl.program_id(0)
    e = embed_ref[...]
    tw = truew_ref[...]
    lf = _read_col(lbl_ref, i).astype(jnp.float32)
    true_expected = _log_uniform_prob(lf) * float(NUM_SAMPLED)
    t = jnp.sum(e * tw, axis=1, keepdims=True) - jnp.log(true_expected)
    m_s = _read_col(ms_ref, i)
    z = _read_col(z_ref, i)
    m = jnp.maximum(m_s, t)
    lse = jnp.log(jnp.exp(t - m) + z * jnp.exp(m_s - m)) + m
    _write_col(out_ref, i, lse - t)


def kernel(softmax_weights, embed, label_idx, zero_bias):
    del zero_bias
    labels = label_idx.reshape(-1)
    lbl_cols = labels.reshape(BATCH // _BLK, _BLK).T

    sampled_ids = _draw_sampled_ids()
    samp_idx_pad = jnp.concatenate(
        [sampled_ids, jnp.zeros((S_PAD - NUM_SAMPLED,), jnp.int32)]
    )
    sid_mask = jnp.concatenate(
        [sampled_ids, jnp.full((S_PAD - NUM_SAMPLED,), -1, jnp.int32)]
    ).reshape(1, S_PAD)
    sampled_expected = _log_uniform_prob(
        sampled_ids.astype(jnp.float32)
    ) * float(NUM_SAMPLED)
    soff = jnp.concatenate(
        [-jnp.log(sampled_expected),
         jnp.full((S_PAD - NUM_SAMPLED,), -1e30, jnp.float32)]
    ).reshape(1, S_PAD)

    samp_w = _sc_gather(softmax_weights, samp_idx_pad, S_PAD, _SAMP_PER_W)
    true_w = _sc_gather(softmax_weights, labels, BATCH, _TRUE_PER_W)

    n_steps = BATCH // _BLK
    grid = (n_steps,)
    m_s, z_s = pl.pallas_call(
        _tc_matmul_body,
        grid=grid,
        in_specs=[
            pl.BlockSpec((_BLK, DIM), lambda i: (i, 0)),
            pl.BlockSpec((S_PAD, DIM), lambda i: (0, 0)),
            pl.BlockSpec((_BLK, BATCH // _BLK), lambda i: (0, 0)),
            pl.BlockSpec((1, S_PAD), lambda i: (0, 0)),
            pl.BlockSpec((1, S_PAD), lambda i: (0, 0)),
        ],
        out_specs=[
            pl.BlockSpec((_BLK, BATCH // _BLK), lambda i: (0, 0)),
            pl.BlockSpec((_BLK, BATCH // _BLK), lambda i: (0, 0)),
        ],
        out_shape=[
            jax.ShapeDtypeStruct((_BLK, n_steps), jnp.float32),
            jax.ShapeDtypeStruct((_BLK, n_steps), jnp.float32),
        ],
    )(embed, samp_w, lbl_cols, sid_mask, soff)

    loss = pl.pallas_call(
        _tc_combine_body,
        grid=grid,
        in_specs=[
            pl.BlockSpec((_BLK, DIM), lambda i: (i, 0)),
            pl.BlockSpec((_BLK, DIM), lambda i: (i, 0)),
            pl.BlockSpec((_BLK, BATCH // _BLK), lambda i: (0, 0)),
            pl.BlockSpec((_BLK, BATCH // _BLK), lambda i: (0, 0)),
            pl.BlockSpec((_BLK, BATCH // _BLK), lambda i: (0, 0)),
        ],
        out_specs=pl.BlockSpec((_BLK, BATCH // _BLK), lambda i: (0, 0)),
        out_shape=jax.ShapeDtypeStruct((_BLK, n_steps), jnp.float32),
    )(embed, true_w, lbl_cols, m_s, z_s)

    return loss.T.reshape(-1)


# speedup vs baseline: 1.1545x; 1.0734x over previous
import functools

import jax
import jax.numpy as jnp
import numpy as np
from jax import lax
from jax.experimental import pallas as pl
from jax.experimental.pallas import tpu as pltpu
from jax.experimental.pallas import tpu_sc as plsc

NUM_CLASSES = 100000
DIM = 128
BATCH = 4096
NUM_SAMPLED = 1000
S_PAD = 1024

_NW = 32
_TRUE_PER_W = BATCH // _NW
_SAMP_PER_W = S_PAD // _NW


def _log_uniform_prob(ids_f):
    return (jnp.log(ids_f + 2.0) - jnp.log(ids_f + 1.0)) / jnp.log(
        float(NUM_CLASSES) + 1.0
    )


def _draw_sampled_ids():
    ks = jax.random.key(42)
    u = jax.random.uniform(ks, (NUM_SAMPLED,), dtype=jnp.float32)
    ids = jnp.floor(jnp.exp(u * jnp.log(float(NUM_CLASSES) + 1.0))) - 1.0
    return jnp.clip(ids, 0, NUM_CLASSES - 1).astype(jnp.int32)


def _sc_gather(table, idx, n_rows, per_w):
    mesh = plsc.VectorSubcoreMesh(core_axis_name="c", subcore_axis_name="s")

    @functools.partial(
        pl.kernel,
        out_type=jax.ShapeDtypeStruct((n_rows, DIM), jnp.float32),
        mesh=mesh,
        scratch_types=(
            pltpu.VMEM((per_w,), jnp.int32),
            pltpu.VMEM((per_w, DIM), jnp.float32),
            pltpu.SemaphoreType.DMA,
        ),
    )
    def gather_kernel(table_hbm, idx_hbm, out_hbm, idx_v, rows_v, sem):
        wid = lax.axis_index("s") * 2 + lax.axis_index("c")
        base = wid * per_w
        pltpu.sync_copy(idx_hbm.at[pl.ds(base, per_w)], idx_v)
        pltpu.async_copy(table_hbm.at[idx_v], rows_v, sem).wait()
        pltpu.sync_copy(rows_v, out_hbm.at[pl.ds(base, per_w)])

    return gather_kernel(table, idx)


_BLK = 1024


def _read_col(ref, i):
    full = ref[...]
    lane = lax.broadcasted_iota(jnp.int32, full.shape, 1)
    return jnp.sum(jnp.where(lane == i, full, 0), axis=1, keepdims=True)


def _write_col(ref, i, v):
    full = ref[...]
    lane = lax.broadcasted_iota(jnp.int32, full.shape, 1)
    ref[...] = jnp.where(lane == i, v, full)


def _tc_matmul_body(embed_ref, sampw_ref, lbl_ref, sid_ref, soff_ref, m_ref, z_ref):
    i = pl.program_id(0)
    e = embed_ref[...]
    sw = sampw_ref[...]
    s = lax.dot_general(
        e, sw, (((1,), (1,)), ((), ())), preferred_element_type=jnp.float32
    )
    s = s + soff_ref[...]
    hit = _read_col(lbl_ref, i) == sid_ref[...]
    s = jnp.where(hit, s - 1e9, s)
    m_s = jnp.max(s, axis=1, keepdims=True)
    _write_col(m_ref, i, m_s)
    _write_col(z_ref, i, jnp.sum(jnp.exp(s - m_s), axis=1, keepdims=True))


def _tc_combine_body(embed_ref, truew_ref, lbl_ref, ms_ref, z_ref, out_ref):
    i = pl.program_id(0)
    e = embed_ref[...]
    tw = truew_ref[...]
    lf = _read_col(lbl_ref, i).astype(jnp.float32)
    true_expected = _log_uniform_prob(lf) * float(NUM_SAMPLED)
    t = jnp.sum(e * tw, axis=1, keepdims=True) - jnp.log(true_expected)
    m_s = _read_col(ms_ref, i)
    z = _read_col(z_ref, i)
    m = jnp.maximum(m_s, t)
    lse = jnp.log(jnp.exp(t - m) + z * jnp.exp(m_s - m)) + m
    out_ref[...] = (lse - t).reshape(_BLK)


def kernel(softmax_weights, embed, label_idx, zero_bias):
    del zero_bias
    labels = label_idx.reshape(-1)
    lbl_cols = labels.reshape(BATCH // _BLK, _BLK).T

    sampled_ids = _draw_sampled_ids()
    samp_idx_pad = jnp.concatenate(
        [sampled_ids, jnp.zeros((S_PAD - NUM_SAMPLED,), jnp.int32)]
    )
    sid_mask = jnp.concatenate(
        [sampled_ids, jnp.full((S_PAD - NUM_SAMPLED,), -1, jnp.int32)]
    ).reshape(1, S_PAD)
    sampled_expected = _log_uniform_prob(
        sampled_ids.astype(jnp.float32)
    ) * float(NUM_SAMPLED)
    soff = jnp.concatenate(
        [-jnp.log(sampled_expected),
         jnp.full((S_PAD - NUM_SAMPLED,), -1e30, jnp.float32)]
    ).reshape(1, S_PAD)

    samp_w = _sc_gather(softmax_weights, samp_idx_pad, S_PAD, _SAMP_PER_W)
    true_w = _sc_gather(softmax_weights, labels, BATCH, _TRUE_PER_W)

    n_steps = BATCH // _BLK
    grid = (n_steps,)
    m_s, z_s = pl.pallas_call(
        _tc_matmul_body,
        grid=grid,
        in_specs=[
            pl.BlockSpec((_BLK, DIM), lambda i: (i, 0)),
            pl.BlockSpec((S_PAD, DIM), lambda i: (0, 0)),
            pl.BlockSpec((_BLK, BATCH // _BLK), lambda i: (0, 0)),
            pl.BlockSpec((1, S_PAD), lambda i: (0, 0)),
            pl.BlockSpec((1, S_PAD), lambda i: (0, 0)),
        ],
        out_specs=[
            pl.BlockSpec((_BLK, BATCH // _BLK), lambda i: (0, 0)),
            pl.BlockSpec((_BLK, BATCH // _BLK), lambda i: (0, 0)),
        ],
        out_shape=[
            jax.ShapeDtypeStruct((_BLK, n_steps), jnp.float32),
            jax.ShapeDtypeStruct((_BLK, n_steps), jnp.float32),
        ],
    )(embed, samp_w, lbl_cols, sid_mask, soff)

    loss = pl.pallas_call(
        _tc_combine_body,
        grid=grid,
        in_specs=[
            pl.BlockSpec((_BLK, DIM), lambda i: (i, 0)),
            pl.BlockSpec((_BLK, DIM), lambda i: (i, 0)),
            pl.BlockSpec((_BLK, BATCH // _BLK), lambda i: (0, 0)),
            pl.BlockSpec((_BLK, BATCH // _BLK), lambda i: (0, 0)),
            pl.BlockSpec((_BLK, BATCH // _BLK), lambda i: (0, 0)),
        ],
        out_specs=pl.BlockSpec((_BLK,), lambda i: (i,)),
        out_shape=jax.ShapeDtypeStruct((BATCH,), jnp.float32),
    )(embed, true_w, lbl_cols, m_s, z_s)

    return loss
